# Initial kernel scaffold; baseline (speedup 1.0000x reference)
import jax
import jax.numpy as jnp
from jax.experimental import pallas as pl


def kernel(hidden_states, gate_w, w1, w2, w3, u1, v1, u2, v2, u3, v3):
    raise NotImplementedError("write your pallas kernel here")


# speedup vs baseline: 2.0234x; 2.0234x over previous
import functools
import jax
import jax.numpy as jnp
from jax.experimental import pallas as pl
from jax.experimental.pallas import tpu as pltpu

E = 8
TOP_K = 2
H = 1024
I = 4096
R = 81
BLK = 256
IB = 1024
NI = I // IB
NB = 40
NPAD = NB * BLK

_INTERPRET = False


def _logits_body(x_ref, gw_ref, out_ref):
    out_ref[...] = jax.lax.dot_general(
        x_ref[...], gw_ref[...], (((1,), (1,)), ((), ())),
        preferred_element_type=jnp.float32)


def _router_logits(x, gate_w):
    T = x.shape[0]
    return pl.pallas_call(
        _logits_body,
        grid=(T // 512,),
        in_specs=[
            pl.BlockSpec((512, H), lambda t: (t, 0)),
            pl.BlockSpec((E, H), lambda t: (0, 0)),
        ],
        out_specs=pl.BlockSpec((512, E), lambda t: (t, 0)),
        out_shape=jax.ShapeDtypeStruct((T, E), jnp.float32),
        interpret=_INTERPRET,
    )(x, gate_w)


def _dgT(a, b):
    return jax.lax.dot_general(
        a, b, (((1,), (1,)), ((), ())), preferred_element_type=jnp.float32)


def _moe_body(be_ref, xs_ref, w1_ref, w3_ref, w2_ref,
              u1_ref, v1_ref, u3_ref, v3_ref, u2_ref, v2_ref, ys_ref):
    i = pl.program_id(1)
    x = xs_ref[...]
    gate = _dgT(x, w1_ref[0]) + _dgT(_dgT(x, v1_ref[0]), u1_ref[0])
    up = _dgT(x, w3_ref[0]) + _dgT(_dgT(x, v3_ref[0]), u3_ref[0])
    h = gate * jax.nn.sigmoid(gate) * up
    part = _dgT(h, w2_ref[0]) + _dgT(_dgT(h, v2_ref[0]), u2_ref[0])

    @pl.when(i == 0)
    def _():
        ys_ref[...] = jnp.zeros_like(ys_ref)

    ys_ref[...] += part


def _grouped_mlp(block_expert, xs, w1, w2, w3, u1, v1, u2, v2, u3, v3):
    grid_spec = pltpu.PrefetchScalarGridSpec(
        num_scalar_prefetch=1,
        grid=(NB, NI),
        in_specs=[
            pl.BlockSpec((BLK, H), lambda b, i, be: (b, 0)),
            pl.BlockSpec((1, IB, H), lambda b, i, be: (be[b], i, 0)),
            pl.BlockSpec((1, IB, H), lambda b, i, be: (be[b], i, 0)),
            pl.BlockSpec((1, H, IB), lambda b, i, be: (be[b], 0, i)),
            pl.BlockSpec((1, IB, R), lambda b, i, be: (be[b], i, 0)),
            pl.BlockSpec((1, R, H), lambda b, i, be: (be[b], 0, 0)),
            pl.BlockSpec((1, IB, R), lambda b, i, be: (be[b], i, 0)),
            pl.BlockSpec((1, R, H), lambda b, i, be: (be[b], 0, 0)),
            pl.BlockSpec((1, H, R), lambda b, i, be: (be[b], 0, 0)),
            pl.BlockSpec((1, R, IB), lambda b, i, be: (be[b], 0, i)),
        ],
        out_specs=pl.BlockSpec((BLK, H), lambda b, i, be: (b, 0)),
    )
    return pl.pallas_call(
        _moe_body,
        grid_spec=grid_spec,
        out_shape=jax.ShapeDtypeStruct((NPAD, H), jnp.float32),
        compiler_params=pltpu.CompilerParams(
            dimension_semantics=("arbitrary", "arbitrary")),
        interpret=_INTERPRET,
    )(block_expert, xs, w1, w3, w2, u1, v1, u3, v3, u2, v2)


def kernel(hidden_states, gate_w, w1, w2, w3, u1, v1, u2, v2, u3, v3):
    b, s, hd = hidden_states.shape
    x = hidden_states.reshape(-1, hd)
    T = x.shape[0]

    logits = _router_logits(x, gate_w)

    probs = jax.nn.softmax(logits, axis=1)
    rw_top, sel = jax.lax.top_k(probs, TOP_K)
    rw_top = rw_top / jnp.sum(rw_top, axis=-1, keepdims=True)

    flat_e = sel.reshape(-1).astype(jnp.int32)
    A = flat_e.shape[0]
    counts = jnp.bincount(flat_e, length=E)
    padded = ((counts + BLK - 1) // BLK) * BLK
    bounds = jnp.cumsum(padded)
    seg_start = bounds - padded
    order = jnp.argsort(flat_e, stable=True)
    sorted_e = flat_e[order]
    cum_counts = jnp.cumsum(counts) - counts
    within = jnp.arange(A) - cum_counts[sorted_e]
    gpos_sorted = seg_start[sorted_e] + within
    pos = jnp.zeros((A,), jnp.int32).at[order].set(gpos_sorted.astype(jnp.int32))

    tok = jnp.arange(A) // TOP_K
    xs = jnp.zeros((NPAD, H), x.dtype).at[pos].set(x[tok])
    block_expert = jnp.minimum(
        jnp.searchsorted(bounds, jnp.arange(NB) * BLK, side='right'),
        E - 1).astype(jnp.int32)

    ys = _grouped_mlp(block_expert, xs, w1, w2, w3, u1, v1, u2, v2, u3, v3)

    inv0 = pos[0::2]
    inv1 = pos[1::2]
    final = ys[inv0] * rw_top[:, 0][:, None] + ys[inv1] * rw_top[:, 1][:, None]
    return final.reshape(b, s, hd), logits
